# Initial kernel scaffold; baseline (speedup 1.0000x reference)
import functools

import jax
import jax.numpy as jnp
from jax import lax
from jax.experimental import pallas as pl
from jax.experimental.pallas import tpu as pltpu
from jax.experimental.pallas import tpu_sc as plsc

BATCH = 16384
SEQ = 50
DIM = 64
NUM_TOK = BATCH * SEQ
NC = 2
NS = 16
NW = NC * NS
PER_W = NUM_TOK // NW
CHUNK = 128
NCH = PER_W // CHUNK
NBUF = 4
NG = NCH // NBUF


def _emb_body(idx_hbm, w_hbm, out_hbm, idx_v, bufs, gsems, ssems):
    wid = lax.axis_index("s") * NC + lax.axis_index("c")
    base = wid * PER_W
    pltpu.sync_copy(idx_hbm.at[wid], idx_v)

    def gather(j, b):
        return pltpu.make_async_copy(w_hbm.at[idx_v.at[j]], bufs[b], gsems[b])

    for b in range(NBUF):
        gather(b, b).start()

    def step(g, _):
        for b in range(NBUF):
            j = g * NBUF + b
            gather(j, b).wait()
            store = pltpu.make_async_copy(
                bufs[b], out_hbm.at[pl.ds(base + j * CHUNK, CHUNK)], ssems[b])
            store.start()

            @pl.when(g < NG - 1)
            def _():
                store.wait()
                gather(j + NBUF, b).start()

        return 0

    lax.fori_loop(0, NG, step, 0)

    for b in range(NBUF):
        pltpu.make_async_copy(
            bufs[b], out_hbm.at[pl.ds(0, CHUNK)], ssems[b]).wait()


def kernel(token_ids, weight):
    idx = token_ids.reshape(NW, NCH, CHUNK).astype(jnp.int32)

    mesh = plsc.VectorSubcoreMesh(core_axis_name="c", subcore_axis_name="s")

    @functools.partial(
        pl.kernel,
        mesh=mesh,
        out_type=jax.ShapeDtypeStruct((NUM_TOK, DIM), jnp.float32),
        scratch_types=[
            pltpu.VMEM((NCH, CHUNK), jnp.int32),
            *[pltpu.VMEM((CHUNK, DIM), jnp.float32) for _ in range(NBUF)],
            *[pltpu.SemaphoreType.DMA for _ in range(2 * NBUF)],
        ],
    )
    def emb(idx_hbm, w_hbm, out_hbm, idx_v, *rest):
        bufs = rest[:NBUF]
        gsems = rest[NBUF:2 * NBUF]
        ssems = rest[2 * NBUF:]
        _emb_body(idx_hbm, w_hbm, out_hbm, idx_v, bufs, gsems, ssems)

    out = emb(idx, weight)
    return out.reshape(BATCH, SEQ, DIM)


# speedup vs baseline: 1.8751x; 1.8751x over previous
import functools

import jax
import jax.numpy as jnp
from jax import lax
from jax.experimental import pallas as pl
from jax.experimental.pallas import tpu as pltpu
from jax.experimental.pallas import tpu_sc as plsc

BATCH = 16384
SEQ = 50
DIM = 64
NUM_TOK = BATCH * SEQ
NC = 2
NS = 16
NW = NC * NS
PER_W = NUM_TOK // NW
CHUNK = 128
NCH = PER_W // CHUNK
NBUF = 4
NG = NCH // NBUF


def _emb_body(idx_hbm, w_hbm, out_hbm, idx_v, bufs, gsems, ssems):
    wid = lax.axis_index("s") * NC + lax.axis_index("c")
    base = wid * PER_W
    pltpu.sync_copy(idx_hbm.at[wid], idx_v)

    def gather(j, b):
        return pltpu.make_async_copy(w_hbm.at[idx_v.at[j]], bufs[b], gsems[b])

    for b in range(NBUF):
        gather(b, b).start()

    def step(g, _):
        for b in range(NBUF):
            j = g * NBUF + b
            gather(j, b).wait()
            store = pltpu.make_async_copy(
                bufs[b], out_hbm.at[pl.ds(base + j * CHUNK, CHUNK)], ssems[b])
            store.start()

            @pl.when(g < NG - 1)
            def _():
                store.wait()
                gather(j + NBUF, b).start()

        return 0

    lax.fori_loop(0, NG, step, 0)

    for b in range(NBUF):
        pltpu.make_async_copy(
            bufs[b], out_hbm.at[pl.ds(0, CHUNK)], ssems[b]).wait()


def kernel(token_ids, weight):
    idx = token_ids.reshape(NW, NCH, CHUNK).astype(jnp.int32)

    mesh = plsc.VectorSubcoreMesh(core_axis_name="c", subcore_axis_name="s")

    @functools.partial(
        pl.kernel,
        mesh=mesh,
        out_type=jax.ShapeDtypeStruct((NUM_TOK, DIM), jnp.float32),
        compiler_params=pltpu.CompilerParams(use_tc_tiling_on_sc=False),
        scratch_types=[
            pltpu.VMEM((NCH, CHUNK), jnp.int32),
            *[pltpu.VMEM((CHUNK, DIM), jnp.float32) for _ in range(NBUF)],
            *[pltpu.SemaphoreType.DMA for _ in range(2 * NBUF)],
        ],
    )
    def emb(idx_hbm, w_hbm, out_hbm, idx_v, *rest):
        bufs = rest[:NBUF]
        gsems = rest[NBUF:2 * NBUF]
        ssems = rest[2 * NBUF:]
        _emb_body(idx_hbm, w_hbm, out_hbm, idx_v, bufs, gsems, ssems)

    out = emb(idx, weight)
    return out.reshape(BATCH, SEQ, DIM)


# speedup vs baseline: 1.8770x; 1.0010x over previous
import functools

import jax
import jax.numpy as jnp
from jax import lax
from jax.experimental import pallas as pl
from jax.experimental.pallas import tpu as pltpu
from jax.experimental.pallas import tpu_sc as plsc

BATCH = 16384
SEQ = 50
DIM = 64
NUM_TOK = BATCH * SEQ
NC = 2
NS = 16
NW = NC * NS
PER_W = NUM_TOK // NW
CHUNK = 128
NCH = PER_W // CHUNK
SUB = 2
ROWS = CHUNK * SUB
NBUF = 4
NSTEP = PER_W // ROWS
NG = NSTEP // NBUF


def _emb_body(idx_hbm, w_hbm, out_hbm, idx_v, bufs, gsems, ssems):
    wid = lax.axis_index("s") * NC + lax.axis_index("c")
    base = wid * PER_W
    pltpu.sync_copy(idx_hbm.at[wid], idx_v)

    def gather(j, b, q):
        return pltpu.make_async_copy(
            w_hbm.at[idx_v.at[j * SUB + q]],
            bufs[b].at[pl.ds(q * CHUNK, CHUNK)],
            gsems[b])

    def fill(j, b):
        for q in range(SUB):
            gather(j, b, q).start()

    def wait_fill(j, b):
        for q in range(SUB):
            gather(j, b, q).wait()

    for b in range(NBUF):
        fill(b, b)

    def step(g, _):
        for b in range(NBUF):
            j = g * NBUF + b
            wait_fill(j, b)
            store = pltpu.make_async_copy(
                bufs[b], out_hbm.at[pl.ds(base + j * ROWS, ROWS)], ssems[b])
            store.start()

            @pl.when(g < NG - 1)
            def _():
                store.wait()
                fill(j + NBUF, b)

        return 0

    lax.fori_loop(0, NG, step, 0)

    for b in range(NBUF):
        pltpu.make_async_copy(
            bufs[b], out_hbm.at[pl.ds(0, ROWS)], ssems[b]).wait()


def kernel(token_ids, weight):
    idx = token_ids.reshape(NW, NCH, CHUNK).astype(jnp.int32)

    mesh = plsc.VectorSubcoreMesh(core_axis_name="c", subcore_axis_name="s")

    @functools.partial(
        pl.kernel,
        mesh=mesh,
        out_type=jax.ShapeDtypeStruct((NUM_TOK, DIM), jnp.float32),
        compiler_params=pltpu.CompilerParams(use_tc_tiling_on_sc=False),
        scratch_types=[
            pltpu.VMEM((NCH, CHUNK), jnp.int32),
            *[pltpu.VMEM((ROWS, DIM), jnp.float32) for _ in range(NBUF)],
            *[pltpu.SemaphoreType.DMA for _ in range(2 * NBUF)],
        ],
    )
    def emb(idx_hbm, w_hbm, out_hbm, idx_v, *rest):
        bufs = rest[:NBUF]
        gsems = rest[NBUF:2 * NBUF]
        ssems = rest[2 * NBUF:]
        _emb_body(idx_hbm, w_hbm, out_hbm, idx_v, bufs, gsems, ssems)

    out = emb(idx, weight)
    return out.reshape(BATCH, SEQ, DIM)
